# baseline (speedup 1.0000x reference)
import numpy as np

import jax
import jax.numpy as jnp
from jax.experimental import pallas as pl
from jax.experimental.pallas import tpu as pltpu

N = 48
P = N * N
NCL = 32
NDL = 16
QT = 384
NQT = 3
NROWS = 40
GID_OFF = 6_000_000


def _pack_columns():
    pairs = [(k, l) for k in range(1, N - 1) for l in range(k, N - 1)]
    kq = np.full(NQT * QT, -1, dtype=np.int32)
    lq = np.full(NQT * QT, -1, dtype=np.int32)
    for c, (k, l) in enumerate(pairs):
        kq[c] = k
        lq[c] = l
    perm = np.where(kq >= 0, kq * N + np.maximum(lq, 0), 0).astype(np.int32)
    padv = (kq >= 0).astype(np.float32)
    tiles = []
    for t in range(NQT):
        sel = slice(QT * t, QT * (t + 1))
        ks = kq[sel][kq[sel] >= 0]
        ls = lq[sel][lq[sel] >= 0]
        irows = N * int(ks.max())
        jrows = P - N * (int(ls.min()) + 1)
        tiles.append(("i", irows) if irows <= jrows else ("j", jrows))
    return kq, lq, perm, padv, tiles


_KQ, _LQ, _PERM, _PADV, _TILES = _pack_columns()
_KQ_R = _KQ.reshape(1, -1)
_LQ_R = _LQ.reshape(1, -1)
_CCNT_C = np.where(_KQ >= 1, _KQ * (N - 1 - _LQ), 0).astype(np.float32).reshape(-1, 1)


def _loss_kernel(consts_ref, cw_ref, dwi_ref, dwj_ref, gwp_ref, kq_ref,
                 lq_ref, ccnt_ref, gw_ref, out_ref):
    qi = pl.program_id(0)

    @pl.when(qi == 0)
    def _init():
        out_ref[0, 0] = 0.0

    for qv, (layout, rws) in enumerate(_TILES):

        @pl.when(qi == qv)
        def _tile(qv=qv, layout=layout, rws=rws):
            gt = gwp_ref[QT * qv:QT * (qv + 1), :]
            mb = jnp.maximum(jnp.max(gt, axis=1, keepdims=True), 0.0)
            eb = jnp.concatenate([jnp.exp(gt - mb), jnp.exp(-mb)], axis=1)

            if layout == "i":
                dt = dwi_ref[:rws, :]
                pv = jax.lax.broadcasted_iota(jnp.int32, (rws, 1), 0)
                ipc = pv // N
                jpc = pv - ipc * N
            else:
                dt = dwj_ref[P - rws:, :]
                pv = (jax.lax.broadcasted_iota(jnp.int32, (rws, 1), 0)
                      + (P - rws))
                jpc = pv // N
                ipc = pv - jpc * N
            ma = jnp.maximum(jnp.max(dt, axis=1, keepdims=True), 0.0)
            ea = jnp.concatenate([jnp.exp(dt - ma), jnp.exp(-ma)], axis=1)

            m = jax.lax.dot_general(
                ea, eb, (((1,), (1,)), ((), ())),
                preferred_element_type=jnp.float32,
                precision=jax.lax.Precision.DEFAULT,
            )

            kqv = kq_ref[0:1, QT * qv:QT * (qv + 1)]
            lqv = lq_ref[0:1, QT * qv:QT * (qv + 1)]
            valid = (ipc < kqv) & (jpc > lqv)
            out_ref[0, 0] += jnp.sum(jnp.where(valid, jnp.log(m), 0.0))

            ccv = ccnt_ref[QT * qv:QT * (qv + 1), :]
            out_ref[0, 0] += jnp.sum(mb * ccv)

    @pl.when(qi == 0)
    def _first_step():
        pva = jax.lax.broadcasted_iota(jnp.int32, (P, 1), 0)
        ipa = pva // N
        jpa = pva - ipa * N
        dwa = dwi_ref[...]
        maa = jnp.maximum(jnp.max(dwa, axis=1, keepdims=True), 0.0)
        mrow = jnp.minimum(jpa - 1, N - 2) - ipa
        rcnt = jnp.where(mrow > 0, mrow * (mrow + 1) // 2, 0).astype(jnp.float32)
        row_term = jnp.sum(maa * rcnt)

        cw = cw_ref[...]
        mc = jnp.maximum(jnp.max(cw, axis=1, keepdims=True), 0.0)
        lse_c = mc + jnp.log(
            jnp.sum(jnp.exp(cw - mc), axis=1, keepdims=True) + jnp.exp(-mc)
        )
        cont_sum = jnp.sum(jnp.where(ipa <= jpa, lse_c, 0.0))

        labs, iis, kks, lls, jjs, gids, isc = [], [], [], [], [], [], []
        for r in range(NROWS):
            lab = consts_ref[r, 0]
            i = consts_ref[r, 1]
            k = consts_ref[r, 2]
            l = consts_ref[r, 3]
            j = consts_ref[r, 4]
            cont = k < 0
            gid = jnp.where(
                cont,
                i * N + j,
                ((i * N + k) * N + l) * N + j + GID_OFF,
            )
            labs.append(lab); iis.append(i); kks.append(k)
            lls.append(l); jjs.append(j); gids.append(gid); isc.append(cont)

        oh32 = jax.lax.broadcasted_iota(jnp.int32, (1, NCL), 1)
        oh16 = jax.lax.broadcasted_iota(jnp.int32, (1, NDL), 1)
        corr = jnp.float32(0.0)
        for r in range(NROWS):
            last = jnp.bool_(True)
            for r2 in range(r + 1, NROWS):
                last = jnp.logical_and(last, gids[r] != gids[r2])
            pidx = iis[r] * N + jjs[r]
            qidx = jnp.where(isc[r], 0, (kks[r] + 1) * N + (lls[r] - 1))
            sel32 = (oh32 == labs[r]).astype(jnp.float32)
            sel16 = (oh16 == labs[r]).astype(jnp.float32)
            vc = jnp.sum(cw_ref[pl.ds(pidx, 1), :] * sel32)
            vd = jnp.sum(
                (dwi_ref[pl.ds(pidx, 1), :] + gw_ref[pl.ds(qidx, 1), :])
                * sel16
            )
            val = jnp.where(isc[r], vc, vd)
            corr = corr + jnp.where(last, val, 0.0)

        out_ref[0, 0] += row_term + cont_sum - corr


@jax.jit
def kernel(cont_weights, disc_weights, gap_weights, constituents):
    cw = cont_weights[0].reshape(P, NCL)
    dwi = disc_weights[0].reshape(P, NDL)
    dwj = disc_weights[0].transpose(1, 0, 2).reshape(P, NDL)
    gw = gap_weights[0].reshape(P, NDL)
    gwp = jnp.take(gw, jnp.asarray(_PERM), axis=0) * jnp.asarray(_PADV)[:, None]
    consts = constituents.astype(jnp.int32)

    full = lambda qi: (0, 0)
    out = pl.pallas_call(
        _loss_kernel,
        grid=(NQT,),
        in_specs=[
            pl.BlockSpec(memory_space=pltpu.SMEM),
            pl.BlockSpec((P, NCL), full),
            pl.BlockSpec((P, NDL), full),
            pl.BlockSpec((P, NDL), full),
            pl.BlockSpec((NQT * QT, NDL), full),
            pl.BlockSpec((1, NQT * QT), full),
            pl.BlockSpec((1, NQT * QT), full),
            pl.BlockSpec((NQT * QT, 1), full),
            pl.BlockSpec((P, NDL), full),
        ],
        out_specs=pl.BlockSpec((1, 1), full, memory_space=pltpu.SMEM),
        out_shape=jax.ShapeDtypeStruct((1, 1), jnp.float32),
        compiler_params=pltpu.CompilerParams(
            dimension_semantics=("arbitrary",),
        ),
    )(consts, cw, dwi, dwj, gwp, jnp.asarray(_KQ_R), jnp.asarray(_LQ_R),
      jnp.asarray(_CCNT_C), gw)
    return out.reshape(1)


# speedup vs baseline: 1.7284x; 1.7284x over previous
import jax
import jax.numpy as jnp
from jax.experimental import pallas as pl
from jax.experimental.pallas import tpu as pltpu

N = 48
P = N * N
NCL = 32
NDL = 16
QT = 768
NQT = P // QT
NROWS = 40
GID_OFF = 6_000_000


def _loss_kernel(consts_ref, cw_ref, dw_ref, gw_ref, out_ref, acc_ref):
    qi = pl.program_id(0)

    @pl.when(qi == 0)
    def _init():
        out_ref[0, 0] = 0.0
        acc_ref[0, 0] = 0.0

    gt = gw_ref[pl.ds(qi * QT, QT), :]
    mb = jnp.maximum(jnp.max(gt, axis=1, keepdims=True), 0.0)
    eb = jnp.concatenate([jnp.exp(gt - mb), jnp.exp(-mb)], axis=1)

    qr = jax.lax.broadcasted_iota(jnp.int32, (1, QT), 1) + qi * QT
    kqr = qr // N
    lqr = qr - kqr * N
    kqs = jnp.where(kqr <= lqr, kqr, -1)

    for qv in range(NQT):

        @pl.when(qi == qv)
        def _tile(qv=qv):
            rws = QT * qv + 720
            dt = dw_ref[:rws, :]
            ma = jnp.maximum(jnp.max(dt, axis=1, keepdims=True), 0.0)
            ea = jnp.concatenate([jnp.exp(dt - ma), jnp.exp(-ma)], axis=1)

            m = jax.lax.dot_general(
                ea, eb, (((1,), (1,)), ((), ())),
                preferred_element_type=jnp.float32,
                precision=jax.lax.Precision.DEFAULT,
            )

            pv = jax.lax.broadcasted_iota(jnp.int32, (rws, 1), 0)
            ipc = pv // N
            jpc = pv - ipc * N
            valid = (ipc < kqs) & (jpc > lqr)
            acc_ref[0, 0] += jnp.sum(jnp.where(valid, jnp.log2(m), 0.0))

    if True:
        qc = jax.lax.broadcasted_iota(jnp.int32, (QT, 1), 0) + qi * QT
        kqc = qc // N
        lqc = qc - kqc * N
        ccnt = jnp.where(kqc <= lqc, kqc * (47 - lqc), 0).astype(jnp.float32)
        out_ref[0, 0] += jnp.sum(mb * ccnt)

        @pl.when(qi == 0)
        def _first_step():
            pva = jax.lax.broadcasted_iota(jnp.int32, (P, 1), 0)
            ipa = pva // N
            jpa = pva - ipa * N
            dwa = dw_ref[...]
            maa = jnp.maximum(jnp.max(dwa, axis=1, keepdims=True), 0.0)
            mrow = jnp.minimum(jpa - 1, 46) - ipa
            rcnt = jnp.where(mrow > 0, mrow * (mrow + 1) // 2, 0).astype(
                jnp.float32
            )
            row_term = jnp.sum(maa * rcnt)

            cw = cw_ref[...]
            mc = jnp.maximum(jnp.max(cw, axis=1, keepdims=True), 0.0)
            lse_c = mc + jnp.log(
                jnp.sum(jnp.exp(cw - mc), axis=1, keepdims=True) + jnp.exp(-mc)
            )
            cont_sum = jnp.sum(jnp.where(ipa <= jpa, lse_c, 0.0))

            labs, iis, kks, lls, jjs, gids, isc = [], [], [], [], [], [], []
            for r in range(NROWS):
                lab = consts_ref[r, 0]
                i = consts_ref[r, 1]
                k = consts_ref[r, 2]
                l = consts_ref[r, 3]
                j = consts_ref[r, 4]
                cont = k < 0
                gid = jnp.where(
                    cont,
                    i * N + j,
                    ((i * N + k) * N + l) * N + j + GID_OFF,
                )
                labs.append(lab); iis.append(i); kks.append(k)
                lls.append(l); jjs.append(j); gids.append(gid); isc.append(cont)

            oh32 = jax.lax.broadcasted_iota(jnp.int32, (1, NCL), 1)
            oh16 = jax.lax.broadcasted_iota(jnp.int32, (1, NDL), 1)
            corr = jnp.float32(0.0)
            for r in range(NROWS):
                last = jnp.bool_(True)
                for r2 in range(r + 1, NROWS):
                    last = jnp.logical_and(last, gids[r] != gids[r2])
                pidx = iis[r] * N + jjs[r]
                qidx = jnp.where(isc[r], 0, (kks[r] + 1) * N + (lls[r] - 1))
                sel32 = (oh32 == labs[r]).astype(jnp.float32)
                sel16 = (oh16 == labs[r]).astype(jnp.float32)
                vc = jnp.sum(cw_ref[pl.ds(pidx, 1), :] * sel32)
                vd = jnp.sum(
                    (dw_ref[pl.ds(pidx, 1), :] + gw_ref[pl.ds(qidx, 1), :])
                    * sel16
                )
                val = jnp.where(isc[r], vc, vd)
                corr = corr + jnp.where(last, val, 0.0)

            out_ref[0, 0] += row_term + cont_sum - corr

    @pl.when(qi == NQT - 1)
    def _final():
        out_ref[0, 0] += acc_ref[0, 0] * jnp.float32(0.6931471805599453)


@jax.jit
def kernel(cont_weights, disc_weights, gap_weights, constituents):
    cw = cont_weights[0].reshape(P, NCL)
    dw = disc_weights[0].reshape(P, NDL)
    gw = gap_weights[0].reshape(P, NDL)
    consts = constituents.astype(jnp.int32)

    out = pl.pallas_call(
        _loss_kernel,
        grid=(NQT,),
        in_specs=[
            pl.BlockSpec(memory_space=pltpu.SMEM),
            pl.BlockSpec((P, NCL), lambda qi: (0, 0)),
            pl.BlockSpec((P, NDL), lambda qi: (0, 0)),
            pl.BlockSpec((P, NDL), lambda qi: (0, 0)),
        ],
        out_specs=pl.BlockSpec(
            (1, 1), lambda qi: (0, 0), memory_space=pltpu.SMEM
        ),
        out_shape=jax.ShapeDtypeStruct((1, 1), jnp.float32),
        compiler_params=pltpu.CompilerParams(
            dimension_semantics=("arbitrary",),
        ),
        scratch_shapes=[pltpu.SMEM((1, 1), jnp.float32)],
    )(consts, cw, dw, gw)
    return out.reshape(1)


# speedup vs baseline: 2.0294x; 1.1741x over previous
import jax
import jax.numpy as jnp
from jax.experimental import pallas as pl
from jax.experimental.pallas import tpu as pltpu

N = 48
P = N * N
NCL = 32
NDL = 16
QT = 768
NQT = P // QT
NROWS = 40
GID_OFF = 6_000_000


def _loss_kernel(consts_ref, cw_ref, dw_ref, gw_ref, out_ref):
    log2_sum = jnp.float32(0.0)
    col_term = jnp.float32(0.0)
    for qv in range(NQT):
        gt = gw_ref[QT * qv:QT * (qv + 1), :]
        mb = jnp.maximum(jnp.max(gt, axis=1, keepdims=True), 0.0)
        eb = jnp.concatenate([jnp.exp(gt - mb), jnp.exp(-mb)], axis=1)

        qr = jax.lax.broadcasted_iota(jnp.int32, (1, QT), 1) + qv * QT
        kqr = qr // N
        lqr = qr - kqr * N
        kqs = jnp.where(kqr <= lqr, kqr, -1)

        rws = QT * qv + 720
        dt = dw_ref[:rws, :]
        ma = jnp.maximum(jnp.max(dt, axis=1, keepdims=True), 0.0)
        ea = jnp.concatenate([jnp.exp(dt - ma), jnp.exp(-ma)], axis=1)

        m = jax.lax.dot_general(
            ea, eb, (((1,), (1,)), ((), ())),
            preferred_element_type=jnp.float32,
            precision=jax.lax.Precision.DEFAULT,
        )

        pv = jax.lax.broadcasted_iota(jnp.int32, (rws, 1), 0)
        ipc = pv // N
        jpc = pv - ipc * N
        valid = (ipc < kqs) & (jpc > lqr)
        log2_sum = log2_sum + jnp.sum(jnp.where(valid, jnp.log2(m), 0.0))

        qc = jax.lax.broadcasted_iota(jnp.int32, (QT, 1), 0) + qv * QT
        kqc = qc // N
        lqc = qc - kqc * N
        ccnt = jnp.where(kqc <= lqc, kqc * (47 - lqc), 0).astype(jnp.float32)
        col_term = col_term + jnp.sum(mb * ccnt)

    if True:
        if True:
            pva = jax.lax.broadcasted_iota(jnp.int32, (P, 1), 0)
            ipa = pva // N
            jpa = pva - ipa * N
            dwa = dw_ref[...]
            maa = jnp.maximum(jnp.max(dwa, axis=1, keepdims=True), 0.0)
            mrow = jnp.minimum(jpa - 1, 46) - ipa
            rcnt = jnp.where(mrow > 0, mrow * (mrow + 1) // 2, 0).astype(
                jnp.float32
            )
            row_term = jnp.sum(maa * rcnt)

            cw = cw_ref[...]
            mc = jnp.maximum(jnp.max(cw, axis=1, keepdims=True), 0.0)
            lse_c = mc + jnp.log(
                jnp.sum(jnp.exp(cw - mc), axis=1, keepdims=True) + jnp.exp(-mc)
            )
            cont_sum = jnp.sum(jnp.where(ipa <= jpa, lse_c, 0.0))

            labs, iis, kks, lls, jjs, gids, isc = [], [], [], [], [], [], []
            for r in range(NROWS):
                lab = consts_ref[r, 0]
                i = consts_ref[r, 1]
                k = consts_ref[r, 2]
                l = consts_ref[r, 3]
                j = consts_ref[r, 4]
                cont = k < 0
                gid = jnp.where(
                    cont,
                    i * N + j,
                    ((i * N + k) * N + l) * N + j + GID_OFF,
                )
                labs.append(lab); iis.append(i); kks.append(k)
                lls.append(l); jjs.append(j); gids.append(gid); isc.append(cont)

            oh32 = jax.lax.broadcasted_iota(jnp.int32, (1, NCL), 1)
            oh16 = jax.lax.broadcasted_iota(jnp.int32, (1, NDL), 1)
            corr = jnp.float32(0.0)
            for r in range(NROWS):
                last = jnp.bool_(True)
                for r2 in range(r + 1, NROWS):
                    last = jnp.logical_and(last, gids[r] != gids[r2])
                pidx = iis[r] * N + jjs[r]
                qidx = jnp.where(isc[r], 0, (kks[r] + 1) * N + (lls[r] - 1))
                sel32 = (oh32 == labs[r]).astype(jnp.float32)
                sel16 = (oh16 == labs[r]).astype(jnp.float32)
                vc = jnp.sum(cw_ref[pl.ds(pidx, 1), :] * sel32)
                vd = jnp.sum(
                    (dw_ref[pl.ds(pidx, 1), :] + gw_ref[pl.ds(qidx, 1), :])
                    * sel16
                )
                val = jnp.where(isc[r], vc, vd)
                corr = corr + jnp.where(last, val, 0.0)

            out_ref[0, 0] = (
                log2_sum * jnp.float32(0.6931471805599453)
                + col_term + row_term + cont_sum - corr
            )


@jax.jit
def kernel(cont_weights, disc_weights, gap_weights, constituents):
    cw = cont_weights[0].reshape(P, NCL)
    dw = disc_weights[0].reshape(P, NDL)
    gw = gap_weights[0].reshape(P, NDL)
    consts = constituents.astype(jnp.int32)

    out = pl.pallas_call(
        _loss_kernel,
        in_specs=[
            pl.BlockSpec(memory_space=pltpu.SMEM),
            pl.BlockSpec((P, NCL), lambda: (0, 0)),
            pl.BlockSpec((P, NDL), lambda: (0, 0)),
            pl.BlockSpec((P, NDL), lambda: (0, 0)),
        ],
        out_specs=pl.BlockSpec(
            (1, 1), lambda: (0, 0), memory_space=pltpu.SMEM
        ),
        out_shape=jax.ShapeDtypeStruct((1, 1), jnp.float32),
    )(consts, cw, dw, gw)
    return out.reshape(1)


# speedup vs baseline: 2.1844x; 1.0763x over previous
import jax
import jax.numpy as jnp
from jax.experimental import pallas as pl
from jax.experimental.pallas import tpu as pltpu

N = 48
P = N * N
NCL = 32
NDL = 16
QT = 768
NQT = P // QT
NROWS = 40
GID_OFF = 6_000_000


def _loss_kernel(consts_ref, cw_ref, dw_ref, gw_ref, out_ref):
    log2_sum = jnp.float32(0.0)
    col_term = jnp.float32(0.0)
    for qv in range(NQT):
        gt = gw_ref[QT * qv:QT * (qv + 1), :]
        mb = jnp.maximum(jnp.max(gt, axis=1, keepdims=True), 0.0)
        eb = jnp.concatenate([jnp.exp(gt - mb), jnp.exp(-mb)], axis=1)

        qr = jax.lax.broadcasted_iota(jnp.int32, (1, QT), 1) + qv * QT
        kqr = qr // N
        lqr = qr - kqr * N
        kqs = jnp.where(kqr <= lqr, kqr, -1)

        if qv < 2:
            rws = QT * qv + 720
            dt = dw_ref[:rws, :]
            pv = jax.lax.broadcasted_iota(jnp.int32, (rws, 1), 0)
            ipc = pv // N
            jpc = pv - ipc * N
        else:
            rws = (N - 33) * N
            dt = (
                dw_ref[...]
                .reshape(N, N, NDL)[:, 33:, :]
                .transpose(1, 0, 2)
                .reshape(rws, NDL)
            )
            pv = jax.lax.broadcasted_iota(jnp.int32, (rws, 1), 0)
            jpc = pv // N + 33
            ipc = pv - (pv // N) * N
        ma = jnp.maximum(jnp.max(dt, axis=1, keepdims=True), 0.0)
        ea = jnp.concatenate([jnp.exp(dt - ma), jnp.exp(-ma)], axis=1)

        m = jax.lax.dot_general(
            ea, eb, (((1,), (1,)), ((), ())),
            preferred_element_type=jnp.float32,
            precision=jax.lax.Precision.DEFAULT,
        )

        valid = (ipc < kqs) & (jpc > lqr)
        log2_sum = log2_sum + jnp.sum(jnp.where(valid, jnp.log2(m), 0.0))

        qc = jax.lax.broadcasted_iota(jnp.int32, (QT, 1), 0) + qv * QT
        kqc = qc // N
        lqc = qc - kqc * N
        ccnt = jnp.where(kqc <= lqc, kqc * (47 - lqc), 0).astype(jnp.float32)
        col_term = col_term + jnp.sum(mb * ccnt)

    if True:
        if True:
            pva = jax.lax.broadcasted_iota(jnp.int32, (P, 1), 0)
            ipa = pva // N
            jpa = pva - ipa * N
            dwa = dw_ref[...]
            maa = jnp.maximum(jnp.max(dwa, axis=1, keepdims=True), 0.0)
            mrow = jnp.minimum(jpa - 1, 46) - ipa
            rcnt = jnp.where(mrow > 0, mrow * (mrow + 1) // 2, 0).astype(
                jnp.float32
            )
            row_term = jnp.sum(maa * rcnt)

            cw = cw_ref[...]
            mc = jnp.maximum(jnp.max(cw, axis=1, keepdims=True), 0.0)
            lse_c = mc + jnp.log(
                jnp.sum(jnp.exp(cw - mc), axis=1, keepdims=True) + jnp.exp(-mc)
            )
            cont_sum = jnp.sum(jnp.where(ipa <= jpa, lse_c, 0.0))

            labs, iis, kks, lls, jjs, gids, isc = [], [], [], [], [], [], []
            for r in range(NROWS):
                lab = consts_ref[r, 0]
                i = consts_ref[r, 1]
                k = consts_ref[r, 2]
                l = consts_ref[r, 3]
                j = consts_ref[r, 4]
                cont = k < 0
                gid = jnp.where(
                    cont,
                    i * N + j,
                    ((i * N + k) * N + l) * N + j + GID_OFF,
                )
                labs.append(lab); iis.append(i); kks.append(k)
                lls.append(l); jjs.append(j); gids.append(gid); isc.append(cont)

            oh32 = jax.lax.broadcasted_iota(jnp.int32, (1, NCL), 1)
            oh16 = jax.lax.broadcasted_iota(jnp.int32, (1, NDL), 1)
            corr = jnp.float32(0.0)
            for r in range(NROWS):
                last = jnp.bool_(True)
                for r2 in range(r + 1, NROWS):
                    last = jnp.logical_and(last, gids[r] != gids[r2])
                pidx = iis[r] * N + jjs[r]
                qidx = jnp.where(isc[r], 0, (kks[r] + 1) * N + (lls[r] - 1))
                sel32 = (oh32 == labs[r]).astype(jnp.float32)
                sel16 = (oh16 == labs[r]).astype(jnp.float32)
                vc = jnp.sum(cw_ref[pl.ds(pidx, 1), :] * sel32)
                vd = jnp.sum(
                    (dw_ref[pl.ds(pidx, 1), :] + gw_ref[pl.ds(qidx, 1), :])
                    * sel16
                )
                val = jnp.where(isc[r], vc, vd)
                corr = corr + jnp.where(last, val, 0.0)

            out_ref[0, 0] = (
                log2_sum * jnp.float32(0.6931471805599453)
                + col_term + row_term + cont_sum - corr
            )


@jax.jit
def kernel(cont_weights, disc_weights, gap_weights, constituents):
    cw = cont_weights[0].reshape(P, NCL)
    dw = disc_weights[0].reshape(P, NDL)
    gw = gap_weights[0].reshape(P, NDL)
    consts = constituents.astype(jnp.int32)

    out = pl.pallas_call(
        _loss_kernel,
        in_specs=[
            pl.BlockSpec(memory_space=pltpu.SMEM),
            pl.BlockSpec((P, NCL), lambda: (0, 0)),
            pl.BlockSpec((P, NDL), lambda: (0, 0)),
            pl.BlockSpec((P, NDL), lambda: (0, 0)),
        ],
        out_specs=pl.BlockSpec(
            (1, 1), lambda: (0, 0), memory_space=pltpu.SMEM
        ),
        out_shape=jax.ShapeDtypeStruct((1, 1), jnp.float32),
    )(consts, cw, dw, gw)
    return out.reshape(1)


# speedup vs baseline: 2.3244x; 1.0641x over previous
import jax
import jax.numpy as jnp
from jax.experimental import pallas as pl
from jax.experimental.pallas import tpu as pltpu

N = 48
P = N * N
NCL = 32
NDL = 16
QT = 384
NQT = P // QT
_TILES = [
    ("i", 336, 0),
    ("i", 720, 0),
    ("i", 1104, 0),
    ("j", 1104, 0),
    ("j", 720, 384),
    ("j", 336, 768),
]
NROWS = 40
GID_OFF = 6_000_000


def _loss_kernel(consts_ref, cw_ref, dw_ref, gw_ref, out_ref):
    log2_sum = jnp.float32(0.0)
    col_term = jnp.float32(0.0)
    dwj = (
        dw_ref[...]
        .reshape(N, N, NDL)[:, 25:, :]
        .transpose(1, 0, 2)
        .reshape((N - 25) * N, NDL)
    )
    for qv, (layout, rws, roff) in enumerate(_TILES):
        gt = gw_ref[QT * qv:QT * (qv + 1), :]
        mb = jnp.maximum(jnp.max(gt, axis=1, keepdims=True), 0.0)
        eb = jnp.concatenate([jnp.exp(gt - mb), jnp.exp(-mb)], axis=1)

        qr = jax.lax.broadcasted_iota(jnp.int32, (1, QT), 1) + qv * QT
        kqr = qr // N
        lqr = qr - kqr * N
        kqs = jnp.where(kqr <= lqr, kqr, -1)

        if layout == "i":
            dt = dw_ref[:rws, :]
            pv = jax.lax.broadcasted_iota(jnp.int32, (rws, 1), 0)
            ipc = pv // N
            jpc = pv - ipc * N
        else:
            dt = dwj[roff:roff + rws, :]
            pv = (jax.lax.broadcasted_iota(jnp.int32, (rws, 1), 0) + roff)
            jpc = pv // N + 25
            ipc = pv - (pv // N) * N
        ma = jnp.maximum(jnp.max(dt, axis=1, keepdims=True), 0.0)
        ea = jnp.concatenate([jnp.exp(dt - ma), jnp.exp(-ma)], axis=1)

        m = jax.lax.dot_general(
            ea, eb, (((1,), (1,)), ((), ())),
            preferred_element_type=jnp.float32,
            precision=jax.lax.Precision.DEFAULT,
        )

        valid = (ipc < kqs) & (jpc > lqr)
        log2_sum = log2_sum + jnp.sum(jnp.where(valid, jnp.log2(m), 0.0))

        qc = jax.lax.broadcasted_iota(jnp.int32, (QT, 1), 0) + qv * QT
        kqc = qc // N
        lqc = qc - kqc * N
        ccnt = jnp.where(kqc <= lqc, kqc * (47 - lqc), 0).astype(jnp.float32)
        col_term = col_term + jnp.sum(mb * ccnt)

    if True:
        if True:
            pva = jax.lax.broadcasted_iota(jnp.int32, (P, 1), 0)
            ipa = pva // N
            jpa = pva - ipa * N
            dwa = dw_ref[...]
            maa = jnp.maximum(jnp.max(dwa, axis=1, keepdims=True), 0.0)
            mrow = jnp.minimum(jpa - 1, 46) - ipa
            rcnt = jnp.where(mrow > 0, mrow * (mrow + 1) // 2, 0).astype(
                jnp.float32
            )
            row_term = jnp.sum(maa * rcnt)

            cw = cw_ref[...]
            mc = jnp.maximum(jnp.max(cw, axis=1, keepdims=True), 0.0)
            lse_c = mc + jnp.log(
                jnp.sum(jnp.exp(cw - mc), axis=1, keepdims=True) + jnp.exp(-mc)
            )
            cont_sum = jnp.sum(jnp.where(ipa <= jpa, lse_c, 0.0))

            labs, iis, kks, lls, jjs, gids, isc = [], [], [], [], [], [], []
            for r in range(NROWS):
                lab = consts_ref[r, 0]
                i = consts_ref[r, 1]
                k = consts_ref[r, 2]
                l = consts_ref[r, 3]
                j = consts_ref[r, 4]
                cont = k < 0
                gid = jnp.where(
                    cont,
                    i * N + j,
                    ((i * N + k) * N + l) * N + j + GID_OFF,
                )
                labs.append(lab); iis.append(i); kks.append(k)
                lls.append(l); jjs.append(j); gids.append(gid); isc.append(cont)

            oh32 = jax.lax.broadcasted_iota(jnp.int32, (1, NCL), 1)
            oh16 = jax.lax.broadcasted_iota(jnp.int32, (1, NDL), 1)
            corr = jnp.float32(0.0)
            for r in range(NROWS):
                last = jnp.bool_(True)
                for r2 in range(r + 1, NROWS):
                    last = jnp.logical_and(last, gids[r] != gids[r2])
                pidx = iis[r] * N + jjs[r]
                qidx = jnp.where(isc[r], 0, (kks[r] + 1) * N + (lls[r] - 1))
                sel32 = (oh32 == labs[r]).astype(jnp.float32)
                sel16 = (oh16 == labs[r]).astype(jnp.float32)
                vc = jnp.sum(cw_ref[pl.ds(pidx, 1), :] * sel32)
                vd = jnp.sum(
                    (dw_ref[pl.ds(pidx, 1), :] + gw_ref[pl.ds(qidx, 1), :])
                    * sel16
                )
                val = jnp.where(isc[r], vc, vd)
                corr = corr + jnp.where(last, val, 0.0)

            out_ref[0, 0] = (
                log2_sum * jnp.float32(0.6931471805599453)
                + col_term + row_term + cont_sum - corr
            )


@jax.jit
def kernel(cont_weights, disc_weights, gap_weights, constituents):
    cw = cont_weights[0].reshape(P, NCL)
    dw = disc_weights[0].reshape(P, NDL)
    gw = gap_weights[0].reshape(P, NDL)
    consts = constituents.astype(jnp.int32)

    out = pl.pallas_call(
        _loss_kernel,
        in_specs=[
            pl.BlockSpec(memory_space=pltpu.SMEM),
            pl.BlockSpec((P, NCL), lambda: (0, 0)),
            pl.BlockSpec((P, NDL), lambda: (0, 0)),
            pl.BlockSpec((P, NDL), lambda: (0, 0)),
        ],
        out_specs=pl.BlockSpec(
            (1, 1), lambda: (0, 0), memory_space=pltpu.SMEM
        ),
        out_shape=jax.ShapeDtypeStruct((1, 1), jnp.float32),
    )(consts, cw, dw, gw)
    return out.reshape(1)


# speedup vs baseline: 2.3359x; 1.0049x over previous
import jax
import jax.numpy as jnp
from jax.experimental import pallas as pl
from jax.experimental.pallas import tpu as pltpu

N = 48
P = N * N
NCL = 32
NDL = 16
QT = 384
NQT = P // QT
_TILES = [
    ("i", 336, 0),
    ("i", 720, 0),
    ("i", 1104, 0),
    ("j", 1104, 0),
    ("j", 720, 384),
    ("j", 336, 768),
]
NROWS = 40
GID_OFF = 6_000_000


def _loss_kernel(consts_ref, cw_ref, dw_ref, gw_ref, out_ref):
    log2_vec = jnp.zeros((1, QT), jnp.float32)
    col_term = jnp.float32(0.0)
    dwj = (
        dw_ref[...]
        .reshape(N, N, NDL)[:, 25:, :]
        .transpose(1, 0, 2)
        .reshape((N - 25) * N, NDL)
    )
    for qv, (layout, rws, roff) in enumerate(_TILES):
        gt = gw_ref[QT * qv:QT * (qv + 1), :]
        mb = jnp.maximum(jnp.max(gt, axis=1, keepdims=True), 0.0)
        eb = jnp.concatenate([jnp.exp(gt - mb), jnp.exp(-mb)], axis=1)

        qr = jax.lax.broadcasted_iota(jnp.int32, (1, QT), 1) + qv * QT
        kqr = qr // N
        lqr = qr - kqr * N
        kqs = jnp.where(kqr <= lqr, kqr, -1)

        if layout == "i":
            dt = dw_ref[:rws, :]
            pv = jax.lax.broadcasted_iota(jnp.int32, (rws, 1), 0)
            ipc = pv // N
            jpc = pv - ipc * N
        else:
            dt = dwj[roff:roff + rws, :]
            pv = (jax.lax.broadcasted_iota(jnp.int32, (rws, 1), 0) + roff)
            jpc = pv // N + 25
            ipc = pv - (pv // N) * N
        ma = jnp.maximum(jnp.max(dt, axis=1, keepdims=True), 0.0)
        ea = jnp.concatenate([jnp.exp(dt - ma), jnp.exp(-ma)], axis=1)

        m = jax.lax.dot_general(
            ea, eb, (((1,), (1,)), ((), ())),
            preferred_element_type=jnp.float32,
            precision=jax.lax.Precision.DEFAULT,
        )

        valid = (ipc < kqs) & (jpc > lqr)
        log2_vec = log2_vec + jnp.sum(
            jnp.where(valid, jnp.log2(m), 0.0), axis=0, keepdims=True
        )

        qc = jax.lax.broadcasted_iota(jnp.int32, (QT, 1), 0) + qv * QT
        kqc = qc // N
        lqc = qc - kqc * N
        ccnt = jnp.where(kqc <= lqc, kqc * (47 - lqc), 0).astype(jnp.float32)
        col_term = col_term + jnp.sum(mb * ccnt)

    if True:
        if True:
            pva = jax.lax.broadcasted_iota(jnp.int32, (P, 1), 0)
            ipa = pva // N
            jpa = pva - ipa * N
            dwa = dw_ref[...]
            maa = jnp.maximum(jnp.max(dwa, axis=1, keepdims=True), 0.0)
            mrow = jnp.minimum(jpa - 1, 46) - ipa
            rcnt = jnp.where(mrow > 0, mrow * (mrow + 1) // 2, 0).astype(
                jnp.float32
            )
            row_term = jnp.sum(maa * rcnt)

            cw = cw_ref[...]
            mc = jnp.maximum(jnp.max(cw, axis=1, keepdims=True), 0.0)
            lse_c = mc + jnp.log(
                jnp.sum(jnp.exp(cw - mc), axis=1, keepdims=True) + jnp.exp(-mc)
            )
            cont_sum = jnp.sum(jnp.where(ipa <= jpa, lse_c, 0.0))

            labs, iis, kks, lls, jjs, gids, isc = [], [], [], [], [], [], []
            for r in range(NROWS):
                lab = consts_ref[r, 0]
                i = consts_ref[r, 1]
                k = consts_ref[r, 2]
                l = consts_ref[r, 3]
                j = consts_ref[r, 4]
                cont = k < 0
                gid = jnp.where(
                    cont,
                    i * N + j,
                    ((i * N + k) * N + l) * N + j + GID_OFF,
                )
                labs.append(lab); iis.append(i); kks.append(k)
                lls.append(l); jjs.append(j); gids.append(gid); isc.append(cont)

            oh32 = jax.lax.broadcasted_iota(jnp.int32, (1, NCL), 1)
            oh16 = jax.lax.broadcasted_iota(jnp.int32, (1, NDL), 1)
            corr = jnp.float32(0.0)
            for r in range(NROWS):
                last = jnp.bool_(True)
                for r2 in range(r + 1, NROWS):
                    last = jnp.logical_and(last, gids[r] != gids[r2])
                pidx = iis[r] * N + jjs[r]
                qidx = jnp.where(isc[r], 0, (kks[r] + 1) * N + (lls[r] - 1))
                sel32 = (oh32 == labs[r]).astype(jnp.float32)
                sel16 = (oh16 == labs[r]).astype(jnp.float32)
                vc = jnp.sum(cw_ref[pl.ds(pidx, 1), :] * sel32)
                vd = jnp.sum(
                    (dw_ref[pl.ds(pidx, 1), :] + gw_ref[pl.ds(qidx, 1), :])
                    * sel16
                )
                val = jnp.where(isc[r], vc, vd)
                corr = corr + jnp.where(last, val, 0.0)

            out_ref[0, 0] = (
                jnp.sum(log2_vec) * jnp.float32(0.6931471805599453)
                + col_term + row_term + cont_sum - corr
            )


@jax.jit
def kernel(cont_weights, disc_weights, gap_weights, constituents):
    cw = cont_weights[0].reshape(P, NCL)
    dw = disc_weights[0].reshape(P, NDL)
    gw = gap_weights[0].reshape(P, NDL)
    consts = constituents.astype(jnp.int32)

    out = pl.pallas_call(
        _loss_kernel,
        in_specs=[
            pl.BlockSpec(memory_space=pltpu.SMEM),
            pl.BlockSpec((P, NCL), lambda: (0, 0)),
            pl.BlockSpec((P, NDL), lambda: (0, 0)),
            pl.BlockSpec((P, NDL), lambda: (0, 0)),
        ],
        out_specs=pl.BlockSpec(
            (1, 1), lambda: (0, 0), memory_space=pltpu.SMEM
        ),
        out_shape=jax.ShapeDtypeStruct((1, 1), jnp.float32),
    )(consts, cw, dw, gw)
    return out.reshape(1)
